# baseline (speedup 1.0000x reference)
import functools

import jax
import jax.numpy as jnp
from jax import lax
from jax.experimental import pallas as pl
from jax.experimental.pallas import tpu as pltpu
from jax.experimental.pallas import tpu_sc as plsc

N = 10000
E = 320000
K = 640000
D = 128

NC = 2
NS = 16
NW = NC * NS

_mesh = plsc.VectorSubcoreMesh(core_axis_name="c", subcore_axis_name="s")


_G_PER_W = E // NW
_G_CH = 1000
_G_NCH = _G_PER_W // _G_CH


@functools.partial(
    pl.kernel,
    out_type=jax.ShapeDtypeStruct((E, D), jnp.float32),
    mesh=_mesh,
    scratch_types=[
        pltpu.VMEM((_G_CH,), jnp.int32),
        pltpu.VMEM((_G_CH, D), jnp.float32),
        pltpu.SemaphoreType.DMA,
    ],
)
def _sc_gather(table_hbm, idx_hbm, out_hbm, idx_v, rows_v, sem):
    wid = lax.axis_index("s") * NC + lax.axis_index("c")
    base = wid * _G_PER_W

    def chunk(j, carry):
        off = base + j * _G_CH
        pltpu.sync_copy(idx_hbm.at[pl.ds(off, _G_CH)], idx_v)
        pltpu.async_copy(table_hbm.at[idx_v], rows_v, sem).wait()
        pltpu.sync_copy(rows_v, out_hbm.at[pl.ds(off, _G_CH)])
        return carry

    lax.fori_loop(0, _G_NCH, chunk, 0)


_S2_PER_W = E // NW
_S2_CH = 200
_S2_NCH = _S2_PER_W // _S2_CH
_S2_NPAD = 10240
_S2_NROWS = _S2_NPAD // NS


@functools.partial(
    pl.kernel,
    out_type=jax.ShapeDtypeStruct((NC, _S2_NPAD, D), jnp.float32),
    mesh=_mesh,
    scratch_types=[
        pltpu.VMEM_SHARED((_S2_NPAD, D), jnp.float32),
        pltpu.VMEM((_S2_CH,), jnp.int32),
        pltpu.VMEM((_S2_CH, D), jnp.float32),
    ],
)
def _sc_scatter_mj(m_hbm, tgt_hbm, zeros_hbm, out_hbm, acc_sh, tvec, mbuf):
    cid = lax.axis_index("c")
    sid = lax.axis_index("s")
    wid = sid * NC + cid
    base = wid * _S2_PER_W
    rbase = sid * _S2_NROWS

    pltpu.sync_copy(zeros_hbm.at[pl.ds(rbase, _S2_NROWS)],
                    acc_sh.at[pl.ds(rbase, _S2_NROWS)])
    plsc.subcore_barrier()

    def chunk(j, carry):
        off = base + j * _S2_CH
        pltpu.sync_copy(tgt_hbm.at[pl.ds(off, _S2_CH)], tvec)
        pltpu.sync_copy(m_hbm.at[pl.ds(off, _S2_CH)], mbuf)
        pltpu.sync_copy(mbuf, acc_sh.at[tvec], add=True)
        return carry

    lax.fori_loop(0, _S2_NCH, chunk, 0)
    plsc.subcore_barrier()
    pltpu.sync_copy(acc_sh.at[pl.ds(rbase, _S2_NROWS)],
                    out_hbm.at[cid, pl.ds(rbase, _S2_NROWS)])


_W = 6400
_WPAD = _W + 128
_WPC = (E // _W) // NC
_SEG = 2000
_KSL = K // NS
_NSEG = _KSL // _SEG
_SB = 32
_ZR = _WPAD // NS
_FR = _W // NS


@functools.partial(
    pl.kernel,
    out_type=(jax.ShapeDtypeStruct((E, D), jnp.float32),
              jax.ShapeDtypeStruct((E, D), jnp.float32)),
    mesh=_mesh,
    compiler_params=pltpu.CompilerParams(needs_layout_passes=False),
    scratch_types=[
        pltpu.VMEM_SHARED((_WPAD, D), jnp.float32),
        pltpu.VMEM_SHARED((_WPAD, D), jnp.float32),
        pltpu.VMEM((_SEG,), jnp.int32),
        pltpu.VMEM((_SEG,), jnp.int32),
        pltpu.VMEM((_SEG + _SB,), jnp.int32),
        pltpu.VMEM((_SEG + _SB,), jnp.int32),
        pltpu.VMEM((_SB,), jnp.int32),
        pltpu.VMEM((_SB,), jnp.int32),
        pltpu.VMEM((_SB,), jnp.int32),
        pltpu.VMEM((_SB, D), jnp.float32),
        pltpu.VMEM((_SB, D), jnp.float32),
        pltpu.VMEM((_SB, D), jnp.float32),
        pltpu.VMEM((_SB, D), jnp.float32),
        pltpu.SemaphoreType.DMA,
    ],
)
def _sc_s1(a_hbm, b_hbm, m_hbm, ns_hbm, n_hbm, z_hbm, s_out, r_out,
           win_s, win_r, s_ch, n_ch, list_s, list_n, a_idx, n_idx, row_idx,
           a_buf, b_buf, m_buf, g_buf, sem):
    cid = lax.axis_index("c")
    sid = lax.axis_index("s")
    kbase = sid * _KSL

    def window(wi, carry_w):
        base = (cid * _WPC + wi) * _W
        pltpu.sync_copy(z_hbm.at[pl.ds(sid * _ZR, _ZR)],
                        win_s.at[pl.ds(sid * _ZR, _ZR)])
        pltpu.sync_copy(z_hbm.at[pl.ds(sid * _ZR, _ZR)],
                        win_r.at[pl.ds(sid * _ZR, _ZR)])
        plsc.subcore_barrier()

        def segment(g, carry_g):
            koff = kbase + g * _SEG
            pltpu.sync_copy(ns_hbm.at[pl.ds(koff, _SEG)], s_ch)
            pltpu.sync_copy(n_hbm.at[pl.ds(koff, _SEG)], n_ch)

            def scan_step(t, cnt):
                s16 = s_ch[pl.ds(t * 16, 16)]
                n16 = n_ch[pl.ds(t * 16, 16)]
                msk = (s16 >= base) & (s16 < base + _W)
                plsc.store_compressed(list_s.at[pl.ds(cnt, 16)], s16, mask=msk)
                plsc.store_compressed(list_n.at[pl.ds(cnt, 16)], n16, mask=msk)
                return cnt + jnp.sum(msk.astype(jnp.int32))

            cnt = lax.fori_loop(0, _SEG // 16, scan_step, 0)
            for q in range(_SB // 16):
                list_s[pl.ds(cnt + q * 16, 16)] = jnp.full((16,), base + _W, jnp.int32)
                list_n[pl.ds(cnt + q * 16, 16)] = jnp.zeros((16,), jnp.int32)
            nb = (cnt + _SB - 1) // _SB

            def drain(bi, carry_b):
                off = bi * _SB
                for q in range(_SB // 16):
                    sv = list_s[pl.ds(off + q * 16, 16)]
                    nv = list_n[pl.ds(off + q * 16, 16)]
                    a_idx[pl.ds(q * 16, 16)] = jnp.minimum(sv, E - 1)
                    row_idx[pl.ds(q * 16, 16)] = sv - base
                    n_idx[pl.ds(q * 16, 16)] = nv
                c1 = pltpu.async_copy(a_hbm.at[a_idx], a_buf, sem)
                c2 = pltpu.async_copy(b_hbm.at[n_idx], b_buf, sem)
                c3 = pltpu.async_copy(m_hbm.at[n_idx], m_buf, sem)
                c1.wait()
                c2.wait()
                c3.wait()

                def row(i, carry_r):
                    for j in range(D // 16):
                        av = a_buf[i, pl.ds(j * 16, 16)]
                        bv = b_buf[i, pl.ds(j * 16, 16)]
                        mv = m_buf[i, pl.ds(j * 16, 16)]
                        g_buf[i, pl.ds(j * 16, 16)] = mv / (1.0 + jnp.exp(-(av + bv)))
                    return carry_r

                lax.fori_loop(0, _SB, row, 0)
                pltpu.sync_copy(m_buf, win_s.at[row_idx], add=True)
                pltpu.sync_copy(g_buf, win_r.at[row_idx], add=True)
                return carry_b

            lax.fori_loop(0, nb, drain, 0)
            return carry_g

        lax.fori_loop(0, _NSEG, segment, 0)
        plsc.subcore_barrier()
        pltpu.sync_copy(win_s.at[pl.ds(sid * _FR, _FR)],
                        s_out.at[pl.ds(base + sid * _FR, _FR)])
        pltpu.sync_copy(win_r.at[pl.ds(sid * _FR, _FR)],
                        r_out.at[pl.ds(base + sid * _FR, _FR)])
        plsc.subcore_barrier()
        return carry_w

    lax.fori_loop(0, _WPC, window, 0)


_BLK = 2000


def _tc2_body(h_src, f_bond, f_mess, wh, wf, bc, wrm, zpre, a_r, wpre, b_r):
    acc = jnp.dot(h_src[...], wh[...], preferred_element_type=jnp.float32)
    acc += jnp.dot(f_bond[...], wf[...], preferred_element_type=jnp.float32)
    acc += bc[...]
    zpre[...] = acc[:, :D]
    a_r[...] = acc[:, D:2 * D]
    wpre[...] = acc[:, 2 * D:]
    b_r[...] = jnp.dot(f_mess[...], wrm[...], preferred_element_type=jnp.float32)


def _tc2(h_src, f_bond, f_mess, wh, wf, bc, wrm):
    grid = (E // _BLK,)
    row_spec = pl.BlockSpec((_BLK, D), lambda i: (i, 0))
    full = lambda shape: pl.BlockSpec(shape, lambda i: (0, 0))
    return pl.pallas_call(
        _tc2_body,
        grid=grid,
        in_specs=[
            row_spec, row_spec, row_spec,
            full((D, 3 * D)), full((D, 3 * D)), full((1, 3 * D)), full((D, D)),
        ],
        out_specs=[row_spec, row_spec, row_spec, row_spec],
        out_shape=[jax.ShapeDtypeStruct((E, D), jnp.float32)] * 4,
    )(h_src, f_bond, f_mess, wh, wf, bc, wrm)


def _tc3_body(zpre, wpre, s, r, wzs, ut, m_new):
    z = jax.nn.sigmoid(zpre[...] + jnp.dot(s[...], wzs[...], preferred_element_type=jnp.float32))
    mn = jnp.tanh(wpre[...] + jnp.dot(r[...], ut[...], preferred_element_type=jnp.float32))
    m_new[...] = (1.0 - z) * s[...] + z * mn


def _tc3(zpre, wpre, s, r, wzs, ut):
    grid = (E // _BLK,)
    row_spec = pl.BlockSpec((_BLK, D), lambda i: (i, 0))
    full = lambda shape: pl.BlockSpec(shape, lambda i: (0, 0))
    return pl.pallas_call(
        _tc3_body,
        grid=grid,
        in_specs=[row_spec, row_spec, row_spec, row_spec, full((D, D)), full((D, D))],
        out_specs=row_spec,
        out_shape=jax.ShapeDtypeStruct((E, D), jnp.float32),
    )(zpre, wpre, s, r, wzs, ut)


_NBLK = 2000


def _tc4_body(f_node, mj, on, om, ob, h_j):
    acc = jnp.dot(f_node[...], on[...], preferred_element_type=jnp.float32)
    mjs = mj[0] + mj[1]
    acc += jnp.dot(mjs, om[...], preferred_element_type=jnp.float32)
    acc += ob[...]
    h_j[...] = acc * 0.5 * (1.0 + lax.erf(acc * (2.0 ** -0.5)))


def _tc4(f_node, mj_parts, on, om, ob):
    grid = (N // _NBLK,)
    row_spec = pl.BlockSpec((_NBLK, D), lambda i: (i, 0))
    mj_spec = pl.BlockSpec((2, _NBLK, D), lambda i: (0, i, 0))
    full = lambda shape: pl.BlockSpec(shape, lambda i: (0,) * len(shape))
    return pl.pallas_call(
        _tc4_body,
        grid=grid,
        in_specs=[row_spec, mj_spec, full((D, D)), full((D, D)), full((1, D))],
        out_specs=row_spec,
        out_shape=jax.ShapeDtypeStruct((N, D), jnp.float32),
    )(f_node, mj_parts, on, om, ob)


def kernel(f_mess, f_node, bond_idx, bond_neibor, f_bond,
           Wz_w, Wz_b, Wr_w, Wr_b, W_w, W_b, U_w, out_w, out_b):
    src = bond_idx[0]
    tgt = bond_idx[1]
    nei_src = bond_neibor[0]
    nei = bond_neibor[1]

    wh = jnp.concatenate([Wz_w[:, :D].T, Wr_w[:, :D].T, W_w[:, :D].T], axis=1)
    wf = jnp.concatenate([Wz_w[:, D:2 * D].T, Wr_w[:, D:2 * D].T, W_w[:, D:].T], axis=1)
    bc = jnp.concatenate([Wz_b, Wr_b, W_b]).reshape(1, 3 * D)
    wrm = Wr_w[:, 2 * D:].T
    wzs = Wz_w[:, 2 * D:].T
    ut = U_w.T
    on = out_w[:, :D].T
    om = out_w[:, D:].T
    ob = out_b.reshape(1, D)

    h_src = _sc_gather(f_node, src)
    zpre, a_r, wpre, b_r = _tc2(h_src, f_bond, f_mess, wh, wf, bc, wrm)

    zeros_w = jnp.zeros((_WPAD, D), jnp.float32)
    s_ij, r_ij = _sc_s1(a_r, b_r, f_mess, nei_src, nei, zeros_w)

    m_new = _tc3(zpre, wpre, s_ij, r_ij, wzs, ut)

    zeros_nd = jnp.zeros((_S2_NPAD, D), jnp.float32)
    mj_parts = _sc_scatter_mj(m_new, tgt, zeros_nd)[:, :N]

    h_j = _tc4(f_node, mj_parts, on, om, ob)
    return (h_j, m_new)


# speedup vs baseline: 1.9289x; 1.9289x over previous
import functools

import jax
import jax.numpy as jnp
from jax import lax
from jax.experimental import pallas as pl
from jax.experimental.pallas import tpu as pltpu
from jax.experimental.pallas import tpu_sc as plsc

N = 10000
E = 320000
K = 640000
D = 128

NC = 2
NS = 16
NW = NC * NS

_mesh = plsc.VectorSubcoreMesh(core_axis_name="c", subcore_axis_name="s")


_G_PER_W = E // NW
_G_CH = 1000
_G_NCH = _G_PER_W // _G_CH


@functools.partial(
    pl.kernel,
    out_type=jax.ShapeDtypeStruct((E, D), jnp.float32),
    mesh=_mesh,
    scratch_types=[
        pltpu.VMEM((_G_CH,), jnp.int32),
        pltpu.VMEM((_G_CH, D), jnp.float32),
        pltpu.SemaphoreType.DMA,
    ],
)
def _sc_gather(table_hbm, idx_hbm, out_hbm, idx_v, rows_v, sem):
    wid = lax.axis_index("s") * NC + lax.axis_index("c")
    base = wid * _G_PER_W

    def chunk(j, carry):
        off = base + j * _G_CH
        pltpu.sync_copy(idx_hbm.at[pl.ds(off, _G_CH)], idx_v)
        pltpu.async_copy(table_hbm.at[idx_v], rows_v, sem).wait()
        pltpu.sync_copy(rows_v, out_hbm.at[pl.ds(off, _G_CH)])
        return carry

    lax.fori_loop(0, _G_NCH, chunk, 0)


_S2_PER_W = E // NW
_S2_CH = 200
_S2_NCH = _S2_PER_W // _S2_CH
_S2_NPAD = 10240
_S2_NROWS = _S2_NPAD // NS


@functools.partial(
    pl.kernel,
    out_type=jax.ShapeDtypeStruct((NC, _S2_NPAD, D), jnp.float32),
    mesh=_mesh,
    scratch_types=[
        pltpu.VMEM_SHARED((_S2_NPAD, D), jnp.float32),
        pltpu.VMEM((_S2_CH,), jnp.int32),
        pltpu.VMEM((_S2_CH, D), jnp.float32),
    ],
)
def _sc_scatter_mj(m_hbm, tgt_hbm, zeros_hbm, out_hbm, acc_sh, tvec, mbuf):
    cid = lax.axis_index("c")
    sid = lax.axis_index("s")
    wid = sid * NC + cid
    base = wid * _S2_PER_W
    rbase = sid * _S2_NROWS

    pltpu.sync_copy(zeros_hbm.at[pl.ds(rbase, _S2_NROWS)],
                    acc_sh.at[pl.ds(rbase, _S2_NROWS)])
    plsc.subcore_barrier()

    def chunk(j, carry):
        off = base + j * _S2_CH
        pltpu.sync_copy(tgt_hbm.at[pl.ds(off, _S2_CH)], tvec)
        pltpu.sync_copy(m_hbm.at[pl.ds(off, _S2_CH)], mbuf)
        pltpu.sync_copy(mbuf, acc_sh.at[tvec], add=True)
        return carry

    lax.fori_loop(0, _S2_NCH, chunk, 0)
    plsc.subcore_barrier()
    pltpu.sync_copy(acc_sh.at[pl.ds(rbase, _S2_NROWS)],
                    out_hbm.at[cid, pl.ds(rbase, _S2_NROWS)])


_W = 3200
_WPAD = _W + 128
_WPC = (E // _W) // NC
_SEG = 4000
_KSL = K // NS
_NSEG = _KSL // _SEG
_SB = 48
_LCAP = 8192
_ZR = _WPAD // NS
_FR = _W // NS


@functools.partial(
    pl.kernel,
    out_type=(jax.ShapeDtypeStruct((E, D), jnp.float32),
              jax.ShapeDtypeStruct((E, D), jnp.float32)),
    mesh=_mesh,
    compiler_params=pltpu.CompilerParams(needs_layout_passes=False),
    scratch_types=[
        pltpu.VMEM_SHARED((_WPAD, D), jnp.float32),
        pltpu.VMEM_SHARED((_WPAD, D), jnp.float32),
        pltpu.VMEM((2 * _SEG,), jnp.int32),
        pltpu.VMEM((2 * _SEG,), jnp.int32),
        pltpu.VMEM((_LCAP,), jnp.int32),
        pltpu.VMEM((_LCAP,), jnp.int32),
        pltpu.VMEM((2, _SB), jnp.int32),
        pltpu.VMEM((2, _SB), jnp.int32),
        pltpu.VMEM((2, _SB), jnp.int32),
        pltpu.VMEM((2, _SB, D), jnp.float32),
        pltpu.VMEM((2, _SB, D), jnp.float32),
        pltpu.VMEM((2, _SB, D), jnp.float32),
        pltpu.VMEM((_SB, D), jnp.float32),
        pltpu.SemaphoreType.DMA,
        pltpu.SemaphoreType.DMA,
    ],
)
def _sc_s1(a_hbm, b_hbm, m_hbm, ns_hbm, n_hbm, z_hbm, s_out, r_out,
           win_s, win_r, s_ch, n_ch, list_s, list_n, a_idx, n_idx, row_idx,
           a_buf, b_buf, m_buf, g_buf, sem_st, sem_g):
    cid = lax.axis_index("c")
    sid = lax.axis_index("s")
    kbase = sid * _KSL

    def stage_issue(g, slot):
        koff = kbase + g * _SEG
        pltpu.async_copy(ns_hbm.at[pl.ds(koff, _SEG)],
                         s_ch.at[pl.ds(slot * _SEG, _SEG)], sem_st)
        pltpu.async_copy(n_hbm.at[pl.ds(koff, _SEG)],
                         n_ch.at[pl.ds(slot * _SEG, _SEG)], sem_st)

    def stage_wait(slot):
        pltpu.make_async_copy(ns_hbm.at[pl.ds(0, _SEG)],
                              s_ch.at[pl.ds(slot * _SEG, _SEG)], sem_st).wait()
        pltpu.make_async_copy(n_hbm.at[pl.ds(0, _SEG)],
                              n_ch.at[pl.ds(slot * _SEG, _SEG)], sem_st).wait()

    def gather_issue(base, bi, slot):
        off = bi * _SB
        for q in range(_SB // 16):
            sv = list_s[pl.ds(off + q * 16, 16)]
            nv = list_n[pl.ds(off + q * 16, 16)]
            a_idx[slot, pl.ds(q * 16, 16)] = jnp.minimum(sv + base, E - 1)
            row_idx[slot, pl.ds(q * 16, 16)] = sv
            n_idx[slot, pl.ds(q * 16, 16)] = nv
        pltpu.async_copy(a_hbm.at[a_idx.at[slot]], a_buf.at[slot], sem_g)
        pltpu.async_copy(b_hbm.at[n_idx.at[slot]], b_buf.at[slot], sem_g)
        pltpu.async_copy(m_hbm.at[n_idx.at[slot]], m_buf.at[slot], sem_g)

    def gather_wait(slot):
        pltpu.make_async_copy(a_hbm.at[a_idx.at[slot]], a_buf.at[slot], sem_g).wait()
        pltpu.make_async_copy(b_hbm.at[n_idx.at[slot]], b_buf.at[slot], sem_g).wait()
        pltpu.make_async_copy(m_hbm.at[n_idx.at[slot]], m_buf.at[slot], sem_g).wait()

    def drain(base, nb):
        @pl.when(nb > 0)
        def _():
            gather_issue(base, 0, 0)

            def body(bi, cur):
                nxt = 1 - cur
                gather_wait(cur)

                @pl.when(bi + 1 < nb)
                def _():
                    gather_issue(base, bi + 1, nxt)

                def row(i, carry_r):
                    for j in range(D // 16):
                        av = a_buf[cur, i, pl.ds(j * 16, 16)]
                        bv = b_buf[cur, i, pl.ds(j * 16, 16)]
                        mv = m_buf[cur, i, pl.ds(j * 16, 16)]
                        g_buf[i, pl.ds(j * 16, 16)] = mv / (1.0 + jnp.exp(-(av + bv)))
                    return carry_r

                lax.fori_loop(0, _SB, row, 0)
                pltpu.sync_copy(m_buf.at[cur], win_s.at[row_idx.at[cur]], add=True)
                pltpu.sync_copy(g_buf, win_r.at[row_idx.at[cur]], add=True)
                return nxt

            lax.fori_loop(0, nb, body, 0)

    def pad_lists(cnt):
        for q in range(_SB // 16):
            list_s[pl.ds(cnt + q * 16, 16)] = jnp.full((16,), _W, jnp.int32)
            list_n[pl.ds(cnt + q * 16, 16)] = jnp.zeros((16,), jnp.int32)

    def window(wi, carry_w):
        base = (cid * _WPC + wi) * _W
        pltpu.sync_copy(z_hbm.at[pl.ds(sid * _ZR, _ZR)],
                        win_s.at[pl.ds(sid * _ZR, _ZR)])
        pltpu.sync_copy(z_hbm.at[pl.ds(sid * _ZR, _ZR)],
                        win_r.at[pl.ds(sid * _ZR, _ZR)])
        plsc.subcore_barrier()
        stage_issue(0, 0)

        def segment(g, cnt_in):
            slot = g % 2
            stage_wait(slot)

            @pl.when(g + 1 < _NSEG)
            def _():
                stage_issue(g + 1, 1 - slot)

            def scan_step(t, cnt):
                s16 = s_ch[pl.ds(slot * _SEG + t * 16, 16)]
                n16 = n_ch[pl.ds(slot * _SEG + t * 16, 16)]
                msk = (s16 >= base) & (s16 < base + _W)
                plsc.store_compressed(list_s.at[pl.ds(cnt, 16)], s16 - base, mask=msk)
                plsc.store_compressed(list_n.at[pl.ds(cnt, 16)], n16, mask=msk)
                return cnt + jnp.sum(msk.astype(jnp.int32))

            cnt = lax.fori_loop(0, _SEG // 16, scan_step, cnt_in)

            def spill(c):
                nfull = c // _SB
                drain(base, nfull)
                rem = c - nfull * _SB
                for q in range(_SB // 16):
                    list_s[pl.ds(q * 16, 16)] = list_s[pl.ds(nfull * _SB + q * 16, 16)]
                    list_n[pl.ds(q * 16, 16)] = list_n[pl.ds(nfull * _SB + q * 16, 16)]
                return rem

            return lax.cond(cnt > _LCAP - _SEG - _SB, spill, lambda c: c, cnt)

        cnt = lax.fori_loop(0, _NSEG, segment, 0)
        pad_lists(cnt)
        drain(base, (cnt + _SB - 1) // _SB)
        plsc.subcore_barrier()
        pltpu.sync_copy(win_s.at[pl.ds(sid * _FR, _FR)],
                        s_out.at[pl.ds(base + sid * _FR, _FR)])
        pltpu.sync_copy(win_r.at[pl.ds(sid * _FR, _FR)],
                        r_out.at[pl.ds(base + sid * _FR, _FR)])
        plsc.subcore_barrier()
        return carry_w

    lax.fori_loop(0, _WPC, window, 0)


_BLK = 2000


def _tc2_body(h_src, f_bond, f_mess, wh, wf, bc, wrm, zpre, a_r, wpre, b_r):
    acc = jnp.dot(h_src[...], wh[...], preferred_element_type=jnp.float32)
    acc += jnp.dot(f_bond[...], wf[...], preferred_element_type=jnp.float32)
    acc += bc[...]
    zpre[...] = acc[:, :D]
    a_r[...] = acc[:, D:2 * D]
    wpre[...] = acc[:, 2 * D:]
    b_r[...] = jnp.dot(f_mess[...], wrm[...], preferred_element_type=jnp.float32)


def _tc2(h_src, f_bond, f_mess, wh, wf, bc, wrm):
    grid = (E // _BLK,)
    row_spec = pl.BlockSpec((_BLK, D), lambda i: (i, 0))
    full = lambda shape: pl.BlockSpec(shape, lambda i: (0, 0))
    return pl.pallas_call(
        _tc2_body,
        grid=grid,
        in_specs=[
            row_spec, row_spec, row_spec,
            full((D, 3 * D)), full((D, 3 * D)), full((1, 3 * D)), full((D, D)),
        ],
        out_specs=[row_spec, row_spec, row_spec, row_spec],
        out_shape=[jax.ShapeDtypeStruct((E, D), jnp.float32)] * 4,
    )(h_src, f_bond, f_mess, wh, wf, bc, wrm)


def _tc3_body(zpre, wpre, s, r, wzs, ut, m_new):
    z = jax.nn.sigmoid(zpre[...] + jnp.dot(s[...], wzs[...], preferred_element_type=jnp.float32))
    mn = jnp.tanh(wpre[...] + jnp.dot(r[...], ut[...], preferred_element_type=jnp.float32))
    m_new[...] = (1.0 - z) * s[...] + z * mn


def _tc3(zpre, wpre, s, r, wzs, ut):
    grid = (E // _BLK,)
    row_spec = pl.BlockSpec((_BLK, D), lambda i: (i, 0))
    full = lambda shape: pl.BlockSpec(shape, lambda i: (0, 0))
    return pl.pallas_call(
        _tc3_body,
        grid=grid,
        in_specs=[row_spec, row_spec, row_spec, row_spec, full((D, D)), full((D, D))],
        out_specs=row_spec,
        out_shape=jax.ShapeDtypeStruct((E, D), jnp.float32),
    )(zpre, wpre, s, r, wzs, ut)


_NBLK = 2000


def _tc4_body(f_node, mj, on, om, ob, h_j):
    acc = jnp.dot(f_node[...], on[...], preferred_element_type=jnp.float32)
    mjs = mj[0] + mj[1]
    acc += jnp.dot(mjs, om[...], preferred_element_type=jnp.float32)
    acc += ob[...]
    h_j[...] = acc * 0.5 * (1.0 + lax.erf(acc * (2.0 ** -0.5)))


def _tc4(f_node, mj_parts, on, om, ob):
    grid = (N // _NBLK,)
    row_spec = pl.BlockSpec((_NBLK, D), lambda i: (i, 0))
    mj_spec = pl.BlockSpec((2, _NBLK, D), lambda i: (0, i, 0))
    full = lambda shape: pl.BlockSpec(shape, lambda i: (0,) * len(shape))
    return pl.pallas_call(
        _tc4_body,
        grid=grid,
        in_specs=[row_spec, mj_spec, full((D, D)), full((D, D)), full((1, D))],
        out_specs=row_spec,
        out_shape=jax.ShapeDtypeStruct((N, D), jnp.float32),
    )(f_node, mj_parts, on, om, ob)


def kernel(f_mess, f_node, bond_idx, bond_neibor, f_bond,
           Wz_w, Wz_b, Wr_w, Wr_b, W_w, W_b, U_w, out_w, out_b):
    src = bond_idx[0]
    tgt = bond_idx[1]
    nei_src = bond_neibor[0]
    nei = bond_neibor[1]

    wh = jnp.concatenate([Wz_w[:, :D].T, Wr_w[:, :D].T, W_w[:, :D].T], axis=1)
    wf = jnp.concatenate([Wz_w[:, D:2 * D].T, Wr_w[:, D:2 * D].T, W_w[:, D:].T], axis=1)
    bc = jnp.concatenate([Wz_b, Wr_b, W_b]).reshape(1, 3 * D)
    wrm = Wr_w[:, 2 * D:].T
    wzs = Wz_w[:, 2 * D:].T
    ut = U_w.T
    on = out_w[:, :D].T
    om = out_w[:, D:].T
    ob = out_b.reshape(1, D)

    h_src = _sc_gather(f_node, src)
    zpre, a_r, wpre, b_r = _tc2(h_src, f_bond, f_mess, wh, wf, bc, wrm)

    zeros_w = jnp.zeros((_WPAD, D), jnp.float32)
    s_ij, r_ij = _sc_s1(a_r, b_r, f_mess, nei_src, nei, zeros_w)

    m_new = _tc3(zpre, wpre, s_ij, r_ij, wzs, ut)

    zeros_nd = jnp.zeros((_S2_NPAD, D), jnp.float32)
    mj_parts = _sc_scatter_mj(m_new, tgt, zeros_nd)[:, :N]

    h_j = _tc4(f_node, mj_parts, on, om, ob)
    return (h_j, m_new)
